# baseline (speedup 1.0000x reference)
import functools

import jax
import jax.numpy as jnp
from jax.experimental import pallas as pl
from jax.experimental.pallas import tpu as pltpu

B, S, D = 1, 2048, 1024
E, TOPK, DFF = 8, 2, 768
T = B * S
BT = 256


def _moe_body(hs_ref, gw_ref, guw_ref, dw_ref, out_ref, logits_ref, acc_ref):
    e = pl.program_id(0)
    t = pl.program_id(1)

    x = hs_ref[...]
    logits = jax.lax.dot_general(
        x, gw_ref[...], (((1,), (1,)), ((), ())),
        preferred_element_type=jnp.float32)
    logits_ref[...] = logits

    iota = jax.lax.broadcasted_iota(jnp.int32, (BT, E), 1)
    m1 = jnp.max(logits, axis=1, keepdims=True)
    idx1 = jnp.min(jnp.where(logits == m1, iota, E), axis=1, keepdims=True)
    l2 = jnp.where(iota == idx1, -jnp.inf, logits)
    m2 = jnp.max(l2, axis=1, keepdims=True)
    idx2 = jnp.min(jnp.where(l2 == m2, iota, E), axis=1, keepdims=True)
    p2 = jnp.exp(m2 - m1)
    denom = 1.0 + p2
    w1 = 1.0 / denom
    w2 = p2 / denom
    we = jnp.where(idx1 == e, w1, jnp.where(idx2 == e, w2, 0.0))

    gu = jax.lax.dot_general(
        x, guw_ref[0], (((1,), (1,)), ((), ())),
        preferred_element_type=jnp.float32)
    g = gu[:, :DFF]
    u = gu[:, DFF:]
    act = g * jax.lax.logistic(g) * u
    y = jax.lax.dot_general(
        act, dw_ref[0], (((1,), (1,)), ((), ())),
        preferred_element_type=jnp.float32)
    contrib = we * y

    sl = pl.ds(t * BT, BT)

    @pl.when(e == 0)
    def _():
        acc_ref[sl, :] = contrib

    @pl.when(e > 0)
    def _():
        acc_ref[sl, :] = acc_ref[sl, :] + contrib

    @pl.when(e == E - 1)
    def _():
        out_ref[...] = acc_ref[sl, :]


@functools.partial(jax.jit, static_argnums=())
def kernel(hidden_states, gate_w, gate_up_w, down_w):
    hs = hidden_states.reshape(T, D)
    grid = (E, T // BT)
    out, logits = pl.pallas_call(
        _moe_body,
        grid=grid,
        in_specs=[
            pl.BlockSpec((BT, D), lambda e, t: (t, 0)),
            pl.BlockSpec((E, D), lambda e, t: (0, 0)),
            pl.BlockSpec((1, 2 * DFF, D), lambda e, t: (e, 0, 0)),
            pl.BlockSpec((1, D, DFF), lambda e, t: (e, 0, 0)),
        ],
        out_specs=[
            pl.BlockSpec((BT, D), lambda e, t: (t, 0)),
            pl.BlockSpec((BT, E), lambda e, t: (t, 0)),
        ],
        out_shape=[
            jax.ShapeDtypeStruct((T, D), jnp.float32),
            jax.ShapeDtypeStruct((T, E), jnp.float32),
        ],
        scratch_shapes=[pltpu.VMEM((T, D), jnp.float32)],
    )(hs, gate_w, gate_up_w, down_w)
    return out.reshape(B, S, D), logits
